# Initial kernel scaffold; baseline (speedup 1.0000x reference)
import jax
import jax.numpy as jnp
from jax.experimental import pallas as pl


def kernel(x, edge_index, labeled_idx, context_idx, W_gcn, b_gcn, W_resize, b_resize):
    raise NotImplementedError("write your pallas kernel here")


# speedup vs baseline: 13.7456x; 13.7456x over previous
import functools

import jax
import jax.numpy as jnp
from jax import lax
from jax.experimental import pallas as pl
from jax.experimental.pallas import tpu as pltpu
from jax.experimental.pallas import tpu_sc as plsc

NN = 10000
EE = 320000
HH = 128
CTX = 5
LL = 4096
NC = 2
NS = 16
NW = NC * NS
CHUNK = 128
CPT = -(-EE // (NW * CHUNK))
EPAD = NW * CPT * CHUNK
NPAD = 10240
RPT = NPAD // NS
GIDX = LL * CTX + LL
GPT = GIDX // NW
GCH = GPT // CHUNK


def _sc_mesh():
    return plsc.VectorSubcoreMesh(
        core_axis_name="c", subcore_axis_name="s", num_cores=NC, num_subcores=NS)


def _sc_degree(dst_flat, ones_src, zeros_h):
    @functools.partial(
        pl.kernel,
        mesh=_sc_mesh(),
        out_type=jax.ShapeDtypeStruct((NC * NPAD, HH), jnp.float32),
        scratch_types=[
            pltpu.VMEM((CHUNK,), jnp.int32),
            pltpu.VMEM((CHUNK, HH), jnp.float32),
            pltpu.VMEM_SHARED((NPAD, HH), jnp.float32),
        ],
    )
    def deg_kernel(dst_hbm, ones_hbm, z_hbm, out_hbm, idx_v, ones_v, acc_sh):
        c = lax.axis_index("c")
        s = lax.axis_index("s")
        pltpu.sync_copy(z_hbm.at[pl.ds(s * RPT, RPT)], acc_sh.at[pl.ds(s * RPT, RPT)])
        pltpu.sync_copy(ones_hbm, ones_v)
        plsc.subcore_barrier()
        base0 = (c * NS + s) * (CPT * CHUNK)

        def body(j, carry):
            pltpu.sync_copy(dst_hbm.at[pl.ds(base0 + j * CHUNK, CHUNK)], idx_v)
            pltpu.sync_copy(ones_v, acc_sh.at[idx_v], add=True)
            return carry

        lax.fori_loop(0, CPT, body, 0)
        plsc.subcore_barrier()
        pltpu.sync_copy(acc_sh.at[pl.ds(s * RPT, RPT)],
                        out_hbm.at[pl.ds(c * NPAD + s * RPT, RPT)])

    return deg_kernel(dst_flat, ones_src, zeros_h)


def _sc_scatter(g, src_flat, dst_flat, zeros_h):
    @functools.partial(
        pl.kernel,
        mesh=_sc_mesh(),
        out_type=jax.ShapeDtypeStruct((NC * NPAD, HH), jnp.float32),
        scratch_types=[
            pltpu.VMEM((CHUNK,), jnp.int32),
            pltpu.VMEM((CHUNK,), jnp.int32),
            pltpu.VMEM((CHUNK, HH), jnp.float32),
            pltpu.VMEM_SHARED((NPAD, HH), jnp.float32),
            pltpu.SemaphoreType.DMA,
        ],
    )
    def msg_kernel(g_hbm, src_hbm, dst_hbm, z_hbm, out_hbm,
                   sidx, didx, rows, acc_sh, sem):
        c = lax.axis_index("c")
        s = lax.axis_index("s")
        pltpu.sync_copy(z_hbm.at[pl.ds(s * RPT, RPT)], acc_sh.at[pl.ds(s * RPT, RPT)])
        plsc.subcore_barrier()
        base0 = (c * NS + s) * (CPT * CHUNK)

        def body(j, carry):
            b = base0 + j * CHUNK
            pltpu.sync_copy(src_hbm.at[pl.ds(b, CHUNK)], sidx)
            pltpu.sync_copy(dst_hbm.at[pl.ds(b, CHUNK)], didx)
            pltpu.async_copy(g_hbm.at[sidx], rows, sem).wait()
            pltpu.sync_copy(rows, acc_sh.at[didx], add=True)
            return carry

        lax.fori_loop(0, CPT, body, 0)
        plsc.subcore_barrier()
        pltpu.sync_copy(acc_sh.at[pl.ds(s * RPT, RPT)],
                        out_hbm.at[pl.ds(c * NPAD + s * RPT, RPT)])

    return msg_kernel(g, src_flat, dst_flat, zeros_h)


def _sc_gather(table, idx_all):
    @functools.partial(
        pl.kernel,
        mesh=_sc_mesh(),
        out_type=jax.ShapeDtypeStruct((GIDX, HH), jnp.float32),
        scratch_types=[
            pltpu.VMEM((CHUNK,), jnp.int32),
            pltpu.VMEM((CHUNK, HH), jnp.float32),
            pltpu.SemaphoreType.DMA,
        ],
    )
    def gather_kernel(tab_hbm, idx_hbm, out_hbm, idx_v, rows_v, sem):
        c = lax.axis_index("c")
        s = lax.axis_index("s")
        base0 = (c * NS + s) * GPT

        def body(j, carry):
            b = base0 + j * CHUNK
            pltpu.sync_copy(idx_hbm.at[pl.ds(b, CHUNK)], idx_v)
            pltpu.async_copy(tab_hbm.at[idx_v], rows_v, sem).wait()
            pltpu.sync_copy(rows_v, out_hbm.at[pl.ds(b, CHUNK)])
            return carry

        lax.fori_loop(0, GCH, body, 0)

    return gather_kernel(table, idx_all)


_ROWS = 1024


def _dot(a, b):
    return jnp.dot(a, b, preferred_element_type=jnp.float32,
                   precision=lax.Precision.HIGHEST)


def _tc_g_dinv(deg2, x_pad, W):
    def body(deg_ref, x_ref, w_ref, g_ref, dv_ref):
        d = deg_ref[0][:, 0:1] + deg_ref[1][:, 0:1] + 1.0
        dinv = lax.rsqrt(d)
        g_ref[...] = _dot(x_ref[...], w_ref[...]) * dinv
        dv_ref[...] = jnp.broadcast_to(dinv, (_ROWS, HH))

    return pl.pallas_call(
        body,
        grid=(NPAD // _ROWS,),
        in_specs=[
            pl.BlockSpec((2, _ROWS, HH), lambda r: (0, r, 0)),
            pl.BlockSpec((_ROWS, HH), lambda r: (r, 0)),
            pl.BlockSpec((HH, HH), lambda r: (0, 0)),
        ],
        out_specs=[pl.BlockSpec((_ROWS, HH), lambda r: (r, 0))] * 2,
        out_shape=[jax.ShapeDtypeStruct((NPAD, HH), jnp.float32)] * 2,
    )(deg2, x_pad, W)


def _tc_layer2(p, g, dinvb, b2, W):
    def body(p_ref, g_ref, dv_ref, b_ref, w_ref, o_ref):
        acc = p_ref[0] + p_ref[1] + g_ref[...]
        fv = jnp.maximum(acc * dv_ref[...] + b_ref[...], 0.0)
        o_ref[...] = _dot(fv, w_ref[...]) * dv_ref[...]

    return pl.pallas_call(
        body,
        grid=(NPAD // _ROWS,),
        in_specs=[
            pl.BlockSpec((2, _ROWS, HH), lambda r: (0, r, 0)),
            pl.BlockSpec((_ROWS, HH), lambda r: (r, 0)),
            pl.BlockSpec((_ROWS, HH), lambda r: (r, 0)),
            pl.BlockSpec((1, HH), lambda r: (0, 0)),
            pl.BlockSpec((HH, HH), lambda r: (0, 0)),
        ],
        out_specs=pl.BlockSpec((_ROWS, HH), lambda r: (r, 0)),
        out_shape=jax.ShapeDtypeStruct((NPAD, HH), jnp.float32),
    )(p, g, dinvb, b2, W)


def _tc_out(p, g2, dinvb, b2):
    def body(p_ref, g_ref, dv_ref, b_ref, o_ref):
        o_ref[...] = ((p_ref[0] + p_ref[1] + g_ref[...]) * dv_ref[...]
                      + b_ref[...])

    return pl.pallas_call(
        body,
        grid=(NPAD // _ROWS,),
        in_specs=[
            pl.BlockSpec((2, _ROWS, HH), lambda r: (0, r, 0)),
            pl.BlockSpec((_ROWS, HH), lambda r: (r, 0)),
            pl.BlockSpec((_ROWS, HH), lambda r: (r, 0)),
            pl.BlockSpec((1, HH), lambda r: (0, 0)),
        ],
        out_specs=pl.BlockSpec((_ROWS, HH), lambda r: (r, 0)),
        out_shape=jax.ShapeDtypeStruct((NPAD, HH), jnp.float32),
    )(p, g2, dinvb, b2)


def _tc_resize(ctx_flat, lab, W_resize, br2):
    R = 1024

    def body(ctx_ref, lab_ref, w_ref, b_ref, o_ref):
        o_ref[...] = lab_ref[...] * (_dot(ctx_ref[...], w_ref[...]) + b_ref[...])

    return pl.pallas_call(
        body,
        grid=(LL // R,),
        in_specs=[
            pl.BlockSpec((R, CTX * HH), lambda r: (r, 0)),
            pl.BlockSpec((R, HH), lambda r: (r, 0)),
            pl.BlockSpec((CTX * HH, HH), lambda r: (0, 0)),
            pl.BlockSpec((1, HH), lambda r: (0, 0)),
        ],
        out_specs=pl.BlockSpec((R, HH), lambda r: (r, 0)),
        out_shape=jax.ShapeDtypeStruct((LL, HH), jnp.float32),
    )(ctx_flat, lab, W_resize, br2)


def kernel(x, edge_index, labeled_idx, context_idx, W_gcn, b_gcn, W_resize, b_resize):
    src = edge_index[0]
    dst = edge_index[1]
    pad_n = EPAD - EE
    fill = NN + (jnp.arange(pad_n, dtype=jnp.int32) % (NPAD - NN))
    src_f = jnp.concatenate([src, fill])
    dst_f = jnp.concatenate([dst, fill])
    x_pad = jnp.pad(x, ((0, NPAD - NN), (0, 0)))
    onesH = jnp.ones((CHUNK, HH), jnp.float32)
    zH = jnp.zeros((NPAD, HH), jnp.float32)
    b2 = b_gcn.reshape(1, HH)
    br2 = b_resize.reshape(1, HH)

    deg = _sc_degree(dst_f, onesH, zH).reshape(NC, NPAD, HH)
    g, dinvb = _tc_g_dinv(deg, x_pad, W_gcn)
    p1 = _sc_scatter(g, src_f, dst_f, zH).reshape(NC, NPAD, HH)
    g2 = _tc_layer2(p1, g, dinvb, b2, W_gcn)
    p2 = _sc_scatter(g2, src_f, dst_f, zH).reshape(NC, NPAD, HH)
    out2 = _tc_out(p2, g2, dinvb, b2)
    idx_all = jnp.concatenate([context_idx.reshape(-1), labeled_idx])
    rows = _sc_gather(out2, idx_all)
    ctx_flat = rows[:LL * CTX].reshape(LL, CTX * HH)
    lab = rows[LL * CTX:]
    return _tc_resize(ctx_flat, lab, W_resize, br2)


# speedup vs baseline: 20.0442x; 1.4582x over previous
import functools

import jax
import jax.numpy as jnp
from jax import lax
from jax.experimental import pallas as pl
from jax.experimental.pallas import tpu as pltpu
from jax.experimental.pallas import tpu_sc as plsc

NN = 10000
EE = 320000
HH = 128
CTX = 5
LL = 4096
NC = 2
NS = 16
NW = NC * NS
CHUNK = 128
CPT = 80
EPAD = NW * CPT * CHUNK
NPAD = 10240
RPT = NPAD // NS
GIDX = LL * CTX + LL
GPT = GIDX // NW
GCH = GPT // CHUNK


def _sc_mesh():
    return plsc.VectorSubcoreMesh(
        core_axis_name="c", subcore_axis_name="s", num_cores=NC, num_subcores=NS)


def _sc_degree(dst_flat, ones_src, zeros_h):
    @functools.partial(
        pl.kernel,
        mesh=_sc_mesh(),
        out_type=jax.ShapeDtypeStruct((NC * NPAD, HH), jnp.float32),
        scratch_types=[
            pltpu.VMEM((CHUNK,), jnp.int32),
            pltpu.VMEM((CHUNK,), jnp.int32),
            pltpu.VMEM((CHUNK, HH), jnp.float32),
            pltpu.VMEM_SHARED((NPAD, HH), jnp.float32),
            pltpu.SemaphoreType.DMA,
            pltpu.SemaphoreType.DMA,
        ],
    )
    def deg_kernel(dst_hbm, ones_hbm, z_hbm, out_hbm, idx0, idx1, ones_v,
                   acc_sh, sem0, sem1):
        c = lax.axis_index("c")
        s = lax.axis_index("s")
        pltpu.sync_copy(z_hbm.at[pl.ds(s * RPT, RPT)], acc_sh.at[pl.ds(s * RPT, RPT)])
        pltpu.sync_copy(ones_hbm, ones_v)
        plsc.subcore_barrier()
        base0 = (c * NS + s) * (CPT * CHUNK)
        idx = [idx0, idx1]
        sems = [sem0, sem1]
        pltpu.async_copy(dst_hbm.at[pl.ds(base0, CHUNK)], idx[0], sems[0])

        def body(g, carry):
            j0 = g * 2
            for b in range(2):
                j = j0 + b
                nb = 1 - b

                @pl.when(j + 1 < CPT)
                def _fire():
                    pltpu.async_copy(
                        dst_hbm.at[pl.ds(base0 + (j + 1) * CHUNK, CHUNK)],
                        idx[nb], sems[nb])

                pltpu.make_async_copy(
                    dst_hbm.at[pl.ds(base0 + j * CHUNK, CHUNK)],
                    idx[b], sems[b]).wait()
                pltpu.sync_copy(ones_v, acc_sh.at[idx[b]], add=True)
            return carry

        lax.fori_loop(0, CPT // 2, body, 0)
        plsc.subcore_barrier()
        pltpu.sync_copy(acc_sh.at[pl.ds(s * RPT, RPT)],
                        out_hbm.at[pl.ds(c * NPAD + s * RPT, RPT)])

    return deg_kernel(dst_flat, ones_src, zeros_h)


def _sc_scatter(g, src_flat, dst_flat, zeros_h):
    @functools.partial(
        pl.kernel,
        mesh=_sc_mesh(),
        out_type=jax.ShapeDtypeStruct((NC * NPAD, HH), jnp.float32),
        scratch_types=[
            pltpu.VMEM((CHUNK,), jnp.int32),
            pltpu.VMEM((CHUNK,), jnp.int32),
            pltpu.VMEM((CHUNK,), jnp.int32),
            pltpu.VMEM((CHUNK,), jnp.int32),
            pltpu.VMEM((CHUNK, HH), jnp.float32),
            pltpu.VMEM((CHUNK, HH), jnp.float32),
            pltpu.VMEM_SHARED((NPAD, HH), jnp.float32),
            pltpu.SemaphoreType.DMA,
            pltpu.SemaphoreType.DMA,
        ],
    )
    def msg_kernel(g_hbm, src_hbm, dst_hbm, z_hbm, out_hbm,
                   sidx0, sidx1, didx0, didx1, rows0, rows1, acc_sh,
                   sem0, sem1):
        c = lax.axis_index("c")
        s = lax.axis_index("s")
        pltpu.sync_copy(z_hbm.at[pl.ds(s * RPT, RPT)], acc_sh.at[pl.ds(s * RPT, RPT)])
        plsc.subcore_barrier()
        base0 = (c * NS + s) * (CPT * CHUNK)
        sidx = [sidx0, sidx1]
        didx = [didx0, didx1]
        rows = [rows0, rows1]
        sems = [sem0, sem1]
        pltpu.sync_copy(src_hbm.at[pl.ds(base0, CHUNK)], sidx[0])
        pltpu.sync_copy(dst_hbm.at[pl.ds(base0, CHUNK)], didx[0])
        pltpu.async_copy(g_hbm.at[sidx[0]], rows[0], sems[0])

        def body(g_, carry):
            j0 = g_ * 2
            for b in range(2):
                j = j0 + b
                nb = 1 - b

                @pl.when(j + 1 < CPT)
                def _fire():
                    bn = base0 + (j + 1) * CHUNK
                    pltpu.sync_copy(src_hbm.at[pl.ds(bn, CHUNK)], sidx[nb])
                    pltpu.sync_copy(dst_hbm.at[pl.ds(bn, CHUNK)], didx[nb])
                    pltpu.async_copy(g_hbm.at[sidx[nb]], rows[nb], sems[nb])

                pltpu.make_async_copy(g_hbm.at[sidx[b]], rows[b], sems[b]).wait()
                pltpu.sync_copy(rows[b], acc_sh.at[didx[b]], add=True)
            return carry

        lax.fori_loop(0, CPT // 2, body, 0)
        plsc.subcore_barrier()
        pltpu.sync_copy(acc_sh.at[pl.ds(s * RPT, RPT)],
                        out_hbm.at[pl.ds(c * NPAD + s * RPT, RPT)])

    return msg_kernel(g, src_flat, dst_flat, zeros_h)


def _sc_gather(table, idx_all):
    @functools.partial(
        pl.kernel,
        mesh=_sc_mesh(),
        out_type=jax.ShapeDtypeStruct((GIDX, HH), jnp.float32),
        scratch_types=[
            pltpu.VMEM((CHUNK,), jnp.int32),
            pltpu.VMEM((CHUNK,), jnp.int32),
            pltpu.VMEM((CHUNK, HH), jnp.float32),
            pltpu.VMEM((CHUNK, HH), jnp.float32),
            pltpu.SemaphoreType.DMA,
            pltpu.SemaphoreType.DMA,
        ],
    )
    def gather_kernel(tab_hbm, idx_hbm, out_hbm, idx0, idx1, rows0, rows1,
                      sem0, sem1):
        c = lax.axis_index("c")
        s = lax.axis_index("s")
        base0 = (c * NS + s) * GPT
        idx = [idx0, idx1]
        rows = [rows0, rows1]
        sems = [sem0, sem1]
        pltpu.sync_copy(idx_hbm.at[pl.ds(base0, CHUNK)], idx[0])
        pltpu.async_copy(tab_hbm.at[idx[0]], rows[0], sems[0])

        def body(g_, carry):
            j0 = g_ * 2
            for b in range(2):
                j = j0 + b
                nb = 1 - b

                @pl.when(j + 1 < GCH)
                def _fire():
                    bn = base0 + (j + 1) * CHUNK
                    pltpu.sync_copy(idx_hbm.at[pl.ds(bn, CHUNK)], idx[nb])
                    pltpu.async_copy(tab_hbm.at[idx[nb]], rows[nb], sems[nb])

                pltpu.make_async_copy(tab_hbm.at[idx[b]], rows[b], sems[b]).wait()
                pltpu.sync_copy(rows[b], out_hbm.at[pl.ds(base0 + j * CHUNK, CHUNK)])
            return carry

        lax.fori_loop(0, GCH // 2, body, 0)

    return gather_kernel(table, idx_all)


_ROWS = 1024


def _dot(a, b):
    return jnp.dot(a, b, preferred_element_type=jnp.float32,
                   precision=lax.Precision.HIGHEST)


def _tc_g_dinv(deg2, x_pad, W):
    def body(deg_ref, x_ref, w_ref, g_ref, dv_ref):
        d = deg_ref[0][:, 0:1] + deg_ref[1][:, 0:1] + 1.0
        dinv = lax.rsqrt(d)
        g_ref[...] = _dot(x_ref[...], w_ref[...]) * dinv
        dv_ref[...] = jnp.broadcast_to(dinv, (_ROWS, HH))

    return pl.pallas_call(
        body,
        grid=(NPAD // _ROWS,),
        in_specs=[
            pl.BlockSpec((2, _ROWS, HH), lambda r: (0, r, 0)),
            pl.BlockSpec((_ROWS, HH), lambda r: (r, 0)),
            pl.BlockSpec((HH, HH), lambda r: (0, 0)),
        ],
        out_specs=[pl.BlockSpec((_ROWS, HH), lambda r: (r, 0))] * 2,
        out_shape=[jax.ShapeDtypeStruct((NPAD, HH), jnp.float32)] * 2,
    )(deg2, x_pad, W)


def _tc_layer2(p, g, dinvb, b2, W):
    def body(p_ref, g_ref, dv_ref, b_ref, w_ref, o_ref):
        acc = p_ref[0] + p_ref[1] + g_ref[...]
        fv = jnp.maximum(acc * dv_ref[...] + b_ref[...], 0.0)
        o_ref[...] = _dot(fv, w_ref[...]) * dv_ref[...]

    return pl.pallas_call(
        body,
        grid=(NPAD // _ROWS,),
        in_specs=[
            pl.BlockSpec((2, _ROWS, HH), lambda r: (0, r, 0)),
            pl.BlockSpec((_ROWS, HH), lambda r: (r, 0)),
            pl.BlockSpec((_ROWS, HH), lambda r: (r, 0)),
            pl.BlockSpec((1, HH), lambda r: (0, 0)),
            pl.BlockSpec((HH, HH), lambda r: (0, 0)),
        ],
        out_specs=pl.BlockSpec((_ROWS, HH), lambda r: (r, 0)),
        out_shape=jax.ShapeDtypeStruct((NPAD, HH), jnp.float32),
    )(p, g, dinvb, b2, W)


def _tc_out(p, g2, dinvb, b2):
    def body(p_ref, g_ref, dv_ref, b_ref, o_ref):
        o_ref[...] = ((p_ref[0] + p_ref[1] + g_ref[...]) * dv_ref[...]
                      + b_ref[...])

    return pl.pallas_call(
        body,
        grid=(NPAD // _ROWS,),
        in_specs=[
            pl.BlockSpec((2, _ROWS, HH), lambda r: (0, r, 0)),
            pl.BlockSpec((_ROWS, HH), lambda r: (r, 0)),
            pl.BlockSpec((_ROWS, HH), lambda r: (r, 0)),
            pl.BlockSpec((1, HH), lambda r: (0, 0)),
        ],
        out_specs=pl.BlockSpec((_ROWS, HH), lambda r: (r, 0)),
        out_shape=jax.ShapeDtypeStruct((NPAD, HH), jnp.float32),
    )(p, g2, dinvb, b2)


def _tc_resize(ctx_flat, lab, W_resize, br2):
    R = 1024

    def body(ctx_ref, lab_ref, w_ref, b_ref, o_ref):
        o_ref[...] = lab_ref[...] * (_dot(ctx_ref[...], w_ref[...]) + b_ref[...])

    return pl.pallas_call(
        body,
        grid=(LL // R,),
        in_specs=[
            pl.BlockSpec((R, CTX * HH), lambda r: (r, 0)),
            pl.BlockSpec((R, HH), lambda r: (r, 0)),
            pl.BlockSpec((CTX * HH, HH), lambda r: (0, 0)),
            pl.BlockSpec((1, HH), lambda r: (0, 0)),
        ],
        out_specs=pl.BlockSpec((R, HH), lambda r: (r, 0)),
        out_shape=jax.ShapeDtypeStruct((LL, HH), jnp.float32),
    )(ctx_flat, lab, W_resize, br2)


def kernel(x, edge_index, labeled_idx, context_idx, W_gcn, b_gcn, W_resize, b_resize):
    src = edge_index[0]
    dst = edge_index[1]
    pad_n = EPAD - EE
    fill = NN + (jnp.arange(pad_n, dtype=jnp.int32) % (NPAD - NN))
    src_f = jnp.concatenate([src, fill])
    dst_f = jnp.concatenate([dst, fill])
    x_pad = jnp.pad(x, ((0, NPAD - NN), (0, 0)))
    onesH = jnp.ones((CHUNK, HH), jnp.float32)
    zH = jnp.zeros((NPAD, HH), jnp.float32)
    b2 = b_gcn.reshape(1, HH)
    br2 = b_resize.reshape(1, HH)

    deg = _sc_degree(dst_f, onesH, zH).reshape(NC, NPAD, HH)
    g, dinvb = _tc_g_dinv(deg, x_pad, W_gcn)
    p1 = _sc_scatter(g, src_f, dst_f, zH).reshape(NC, NPAD, HH)
    g2 = _tc_layer2(p1, g, dinvb, b2, W_gcn)
    p2 = _sc_scatter(g2, src_f, dst_f, zH).reshape(NC, NPAD, HH)
    out2 = _tc_out(p2, g2, dinvb, b2)
    idx_all = jnp.concatenate([context_idx.reshape(-1), labeled_idx])
    rows = _sc_gather(out2, idx_all)
    ctx_flat = rows[:LL * CTX].reshape(LL, CTX * HH)
    lab = rows[LL * CTX:]
    return _tc_resize(ctx_flat, lab, W_resize, br2)


# speedup vs baseline: 20.0455x; 1.0001x over previous
import functools

import jax
import jax.numpy as jnp
from jax import lax
from jax.experimental import pallas as pl
from jax.experimental.pallas import tpu as pltpu
from jax.experimental.pallas import tpu_sc as plsc

NN = 10000
EE = 320000
HH = 128
CTX = 5
LL = 4096
NC = 2
NS = 16
NW = NC * NS
CHUNK = 128
CPT = 80
EPAD = NW * CPT * CHUNK
NPAD = 10240
RPT = NPAD // NS
GIDX = LL * CTX + LL
GPT = GIDX // NW
GCH = GPT // CHUNK


def _sc_mesh():
    return plsc.VectorSubcoreMesh(
        core_axis_name="c", subcore_axis_name="s", num_cores=NC, num_subcores=NS)


DW = HH


def _sc_degree(dst_flat, ones_row, zeros_nd):
    @functools.partial(
        pl.kernel,
        mesh=_sc_mesh(),
        out_type=jax.ShapeDtypeStruct((NC * NPAD, DW), jnp.float32),
        scratch_types=[
            pltpu.VMEM((CHUNK,), jnp.int32),
            pltpu.VMEM((CHUNK,), jnp.int32),
            pltpu.VMEM((CHUNK, DW), jnp.float32),
            pltpu.VMEM_SHARED((NPAD, DW), jnp.float32),
            pltpu.SemaphoreType.DMA,
            pltpu.SemaphoreType.DMA,
        ],
    )
    def deg_kernel(dst_hbm, ones_hbm, z_hbm, out_hbm, idx0, idx1, ones_v,
                   acc_sh, sem0, sem1):
        c = lax.axis_index("c")
        s = lax.axis_index("s")
        pltpu.sync_copy(z_hbm.at[pl.ds(s * RPT, RPT)],
                        acc_sh.at[pl.ds(s * RPT, RPT)])
        pltpu.sync_copy(ones_hbm, ones_v)
        plsc.subcore_barrier()
        base0 = (c * NS + s) * (CPT * CHUNK)
        idx = [idx0, idx1]
        sems = [sem0, sem1]
        pltpu.async_copy(dst_hbm.at[pl.ds(base0, CHUNK)], idx[0], sems[0])

        def body(g, carry):
            j0 = g * 2
            for b in range(2):
                j = j0 + b
                nb = 1 - b

                @pl.when(j + 1 < CPT)
                def _fire():
                    pltpu.async_copy(
                        dst_hbm.at[pl.ds(base0 + (j + 1) * CHUNK, CHUNK)],
                        idx[nb], sems[nb])

                pltpu.make_async_copy(
                    dst_hbm.at[pl.ds(base0 + j * CHUNK, CHUNK)],
                    idx[b], sems[b]).wait()
                pltpu.sync_copy(ones_v, acc_sh.at[idx[b]], add=True)
            return carry

        lax.fori_loop(0, CPT // 2, body, 0)
        plsc.subcore_barrier()
        pltpu.sync_copy(acc_sh.at[pl.ds(s * RPT, RPT)],
                        out_hbm.at[pl.ds(c * NPAD + s * RPT, RPT)])

    return deg_kernel(dst_flat, ones_row, zeros_nd)


def _sc_scatter(g, src_flat, dst_flat, zeros_h):
    @functools.partial(
        pl.kernel,
        mesh=_sc_mesh(),
        out_type=jax.ShapeDtypeStruct((NC * NPAD, HH), jnp.float32),
        scratch_types=[
            pltpu.VMEM((CHUNK,), jnp.int32),
            pltpu.VMEM((CHUNK,), jnp.int32),
            pltpu.VMEM((CHUNK,), jnp.int32),
            pltpu.VMEM((CHUNK,), jnp.int32),
            pltpu.VMEM((CHUNK, HH), jnp.float32),
            pltpu.VMEM((CHUNK, HH), jnp.float32),
            pltpu.VMEM_SHARED((NPAD, HH), jnp.float32),
            pltpu.SemaphoreType.DMA,
            pltpu.SemaphoreType.DMA,
        ],
    )
    def msg_kernel(g_hbm, src_hbm, dst_hbm, z_hbm, out_hbm,
                   sidx0, sidx1, didx0, didx1, rows0, rows1, acc_sh,
                   sem0, sem1):
        c = lax.axis_index("c")
        s = lax.axis_index("s")
        pltpu.sync_copy(z_hbm.at[pl.ds(s * RPT, RPT)], acc_sh.at[pl.ds(s * RPT, RPT)])
        plsc.subcore_barrier()
        base0 = (c * NS + s) * (CPT * CHUNK)
        sidx = [sidx0, sidx1]
        didx = [didx0, didx1]
        rows = [rows0, rows1]
        sems = [sem0, sem1]
        pltpu.sync_copy(src_hbm.at[pl.ds(base0, CHUNK)], sidx[0])
        pltpu.sync_copy(dst_hbm.at[pl.ds(base0, CHUNK)], didx[0])
        pltpu.async_copy(g_hbm.at[sidx[0]], rows[0], sems[0])

        def body(g_, carry):
            j0 = g_ * 2
            for b in range(2):
                j = j0 + b
                nb = 1 - b

                @pl.when(j + 1 < CPT)
                def _fire():
                    bn = base0 + (j + 1) * CHUNK
                    pltpu.sync_copy(src_hbm.at[pl.ds(bn, CHUNK)], sidx[nb])
                    pltpu.sync_copy(dst_hbm.at[pl.ds(bn, CHUNK)], didx[nb])
                    pltpu.async_copy(g_hbm.at[sidx[nb]], rows[nb], sems[nb])

                pltpu.make_async_copy(g_hbm.at[sidx[b]], rows[b], sems[b]).wait()
                pltpu.sync_copy(rows[b], acc_sh.at[didx[b]], add=True)
            return carry

        lax.fori_loop(0, CPT // 2, body, 0)
        plsc.subcore_barrier()
        pltpu.sync_copy(acc_sh.at[pl.ds(s * RPT, RPT)],
                        out_hbm.at[pl.ds(c * NPAD + s * RPT, RPT)])

    return msg_kernel(g, src_flat, dst_flat, zeros_h)


def _sc_gather(table, idx_all):
    @functools.partial(
        pl.kernel,
        mesh=_sc_mesh(),
        out_type=jax.ShapeDtypeStruct((GIDX, HH), jnp.float32),
        scratch_types=[
            pltpu.VMEM((CHUNK,), jnp.int32),
            pltpu.VMEM((CHUNK,), jnp.int32),
            pltpu.VMEM((CHUNK, HH), jnp.float32),
            pltpu.VMEM((CHUNK, HH), jnp.float32),
            pltpu.SemaphoreType.DMA,
            pltpu.SemaphoreType.DMA,
        ],
    )
    def gather_kernel(tab_hbm, idx_hbm, out_hbm, idx0, idx1, rows0, rows1,
                      sem0, sem1):
        c = lax.axis_index("c")
        s = lax.axis_index("s")
        base0 = (c * NS + s) * GPT
        idx = [idx0, idx1]
        rows = [rows0, rows1]
        sems = [sem0, sem1]
        pltpu.sync_copy(idx_hbm.at[pl.ds(base0, CHUNK)], idx[0])
        pltpu.async_copy(tab_hbm.at[idx[0]], rows[0], sems[0])

        def body(g_, carry):
            j0 = g_ * 2
            for b in range(2):
                j = j0 + b
                nb = 1 - b

                @pl.when(j + 1 < GCH)
                def _fire():
                    bn = base0 + (j + 1) * CHUNK
                    pltpu.sync_copy(idx_hbm.at[pl.ds(bn, CHUNK)], idx[nb])
                    pltpu.async_copy(tab_hbm.at[idx[nb]], rows[nb], sems[nb])

                pltpu.make_async_copy(tab_hbm.at[idx[b]], rows[b], sems[b]).wait()
                pltpu.sync_copy(rows[b], out_hbm.at[pl.ds(base0 + j * CHUNK, CHUNK)])
            return carry

        lax.fori_loop(0, GCH // 2, body, 0)

    return gather_kernel(table, idx_all)


_ROWS = 1024


def _dot(a, b):
    return jnp.dot(a, b, preferred_element_type=jnp.float32,
                   precision=lax.Precision.HIGHEST)


def _tc_g_dinv(deg_pair, x_pad, W):
    def body(deg_ref, x_ref, w_ref, g_ref, dv_ref):
        dinv = lax.rsqrt(deg_ref[0] + deg_ref[1] + 1.0)
        g_ref[...] = _dot(x_ref[...], w_ref[...]) * dinv
        dv_ref[...] = dinv

    return pl.pallas_call(
        body,
        grid=(NPAD // _ROWS,),
        in_specs=[
            pl.BlockSpec((2, _ROWS, DW), lambda r: (0, r, 0)),
            pl.BlockSpec((_ROWS, HH), lambda r: (r, 0)),
            pl.BlockSpec((HH, HH), lambda r: (0, 0)),
        ],
        out_specs=[pl.BlockSpec((_ROWS, HH), lambda r: (r, 0))] * 2,
        out_shape=[jax.ShapeDtypeStruct((NPAD, HH), jnp.float32)] * 2,
    )(deg_pair, x_pad, W)


def _tc_layer2(p, g, dinvb, b2, W):
    def body(p_ref, g_ref, dv_ref, b_ref, w_ref, o_ref):
        acc = p_ref[0] + p_ref[1] + g_ref[...]
        fv = jnp.maximum(acc * dv_ref[...] + b_ref[...], 0.0)
        o_ref[...] = _dot(fv, w_ref[...]) * dv_ref[...]

    return pl.pallas_call(
        body,
        grid=(NPAD // _ROWS,),
        in_specs=[
            pl.BlockSpec((2, _ROWS, HH), lambda r: (0, r, 0)),
            pl.BlockSpec((_ROWS, HH), lambda r: (r, 0)),
            pl.BlockSpec((_ROWS, HH), lambda r: (r, 0)),
            pl.BlockSpec((1, HH), lambda r: (0, 0)),
            pl.BlockSpec((HH, HH), lambda r: (0, 0)),
        ],
        out_specs=pl.BlockSpec((_ROWS, HH), lambda r: (r, 0)),
        out_shape=jax.ShapeDtypeStruct((NPAD, HH), jnp.float32),
    )(p, g, dinvb, b2, W)


def _tc_out(p, g2, dinvb, b2):
    def body(p_ref, g_ref, dv_ref, b_ref, o_ref):
        o_ref[...] = ((p_ref[0] + p_ref[1] + g_ref[...]) * dv_ref[...]
                      + b_ref[...])

    return pl.pallas_call(
        body,
        grid=(NPAD // _ROWS,),
        in_specs=[
            pl.BlockSpec((2, _ROWS, HH), lambda r: (0, r, 0)),
            pl.BlockSpec((_ROWS, HH), lambda r: (r, 0)),
            pl.BlockSpec((_ROWS, HH), lambda r: (r, 0)),
            pl.BlockSpec((1, HH), lambda r: (0, 0)),
        ],
        out_specs=pl.BlockSpec((_ROWS, HH), lambda r: (r, 0)),
        out_shape=jax.ShapeDtypeStruct((NPAD, HH), jnp.float32),
    )(p, g2, dinvb, b2)


def _tc_resize(ctx_flat, lab, W_resize, br2):
    R = 1024

    def body(ctx_ref, lab_ref, w_ref, b_ref, o_ref):
        o_ref[...] = lab_ref[...] * (_dot(ctx_ref[...], w_ref[...]) + b_ref[...])

    return pl.pallas_call(
        body,
        grid=(LL // R,),
        in_specs=[
            pl.BlockSpec((R, CTX * HH), lambda r: (r, 0)),
            pl.BlockSpec((R, HH), lambda r: (r, 0)),
            pl.BlockSpec((CTX * HH, HH), lambda r: (0, 0)),
            pl.BlockSpec((1, HH), lambda r: (0, 0)),
        ],
        out_specs=pl.BlockSpec((R, HH), lambda r: (r, 0)),
        out_shape=jax.ShapeDtypeStruct((LL, HH), jnp.float32),
    )(ctx_flat, lab, W_resize, br2)


def kernel(x, edge_index, labeled_idx, context_idx, W_gcn, b_gcn, W_resize, b_resize):
    src = edge_index[0]
    dst = edge_index[1]
    pad_n = EPAD - EE
    fill = NN + (jnp.arange(pad_n, dtype=jnp.int32) % (NPAD - NN))
    src_f = jnp.concatenate([src, fill])
    dst_f = jnp.concatenate([dst, fill])
    x_pad = jnp.pad(x, ((0, NPAD - NN), (0, 0)))
    zH = jnp.zeros((NPAD, HH), jnp.float32)
    zND = jnp.zeros((NPAD, DW), jnp.float32)
    onesD = jnp.ones((CHUNK, DW), jnp.float32)
    b2 = b_gcn.reshape(1, HH)
    br2 = b_resize.reshape(1, HH)

    deg_pair = _sc_degree(dst_f, onesD, zND).reshape(NC, NPAD, DW)
    g, dinvb = _tc_g_dinv(deg_pair, x_pad, W_gcn)
    p1 = _sc_scatter(g, src_f, dst_f, zH).reshape(NC, NPAD, HH)
    g2 = _tc_layer2(p1, g, dinvb, b2, W_gcn)
    p2 = _sc_scatter(g2, src_f, dst_f, zH).reshape(NC, NPAD, HH)
    out2 = _tc_out(p2, g2, dinvb, b2)
    idx_all = jnp.concatenate([context_idx.reshape(-1), labeled_idx])
    rows = _sc_gather(out2, idx_all)
    ctx_flat = rows[:LL * CTX].reshape(LL, CTX * HH)
    lab = rows[LL * CTX:]
    return _tc_resize(ctx_flat, lab, W_resize, br2)


# speedup vs baseline: 24.1899x; 1.2067x over previous
import functools

import jax
import jax.numpy as jnp
from jax import lax
from jax.experimental import pallas as pl
from jax.experimental.pallas import tpu as pltpu
from jax.experimental.pallas import tpu_sc as plsc

NN = 10000
EE = 320000
HH = 128
CTX = 5
LL = 4096
NC = 2
NS = 16
NW = NC * NS
CHUNK = 128
CPT = 80
EPAD = NW * CPT * CHUNK
NPAD = 10240
RPT = NPAD // NS
GIDX = LL * CTX + LL
GPT = GIDX // NW
GCH = GPT // CHUNK


def _sc_mesh():
    return plsc.VectorSubcoreMesh(
        core_axis_name="c", subcore_axis_name="s", num_cores=NC, num_subcores=NS)


DW = HH


def _sc_degree(dst_flat, ones_row, zeros_nd):
    @functools.partial(
        pl.kernel,
        mesh=_sc_mesh(),
        out_type=jax.ShapeDtypeStruct((NC * NPAD, DW), jnp.float32),
        scratch_types=[
            pltpu.VMEM((CHUNK,), jnp.int32),
            pltpu.VMEM((CHUNK,), jnp.int32),
            pltpu.VMEM((CHUNK, DW), jnp.float32),
            pltpu.VMEM_SHARED((NPAD, DW), jnp.float32),
            pltpu.SemaphoreType.DMA,
            pltpu.SemaphoreType.DMA,
        ],
    )
    def deg_kernel(dst_hbm, ones_hbm, z_hbm, out_hbm, idx0, idx1, ones_v,
                   acc_sh, sem0, sem1):
        c = lax.axis_index("c")
        s = lax.axis_index("s")
        pltpu.sync_copy(z_hbm.at[pl.ds(s * RPT, RPT)],
                        acc_sh.at[pl.ds(s * RPT, RPT)])
        pltpu.sync_copy(ones_hbm, ones_v)
        plsc.subcore_barrier()
        base0 = (c * NS + s) * (CPT * CHUNK)
        idx = [idx0, idx1]
        sems = [sem0, sem1]
        pltpu.async_copy(dst_hbm.at[pl.ds(base0, CHUNK)], idx[0], sems[0])

        def body(g, carry):
            j0 = g * 2
            for b in range(2):
                j = j0 + b
                nb = 1 - b

                @pl.when(j + 1 < CPT)
                def _fire():
                    pltpu.async_copy(
                        dst_hbm.at[pl.ds(base0 + (j + 1) * CHUNK, CHUNK)],
                        idx[nb], sems[nb])

                pltpu.make_async_copy(
                    dst_hbm.at[pl.ds(base0 + j * CHUNK, CHUNK)],
                    idx[b], sems[b]).wait()
                pltpu.sync_copy(ones_v, acc_sh.at[idx[b]], add=True)
            return carry

        lax.fori_loop(0, CPT // 2, body, 0)
        plsc.subcore_barrier()
        pltpu.sync_copy(acc_sh.at[pl.ds(s * RPT, RPT)],
                        out_hbm.at[pl.ds(c * NPAD + s * RPT, RPT)])

    return deg_kernel(dst_flat, ones_row, zeros_nd)


_SRING = 4


def _sc_scatter(g, src_flat, dst2d, zeros_h):
    @functools.partial(
        pl.kernel,
        mesh=_sc_mesh(),
        out_type=jax.ShapeDtypeStruct((NC * NPAD, HH), jnp.float32),
        scratch_types=[
            pltpu.VMEM((CPT, CHUNK), jnp.int32),
        ] + [pltpu.VMEM((CHUNK,), jnp.int32)] * _SRING + [
            pltpu.VMEM((CHUNK, HH), jnp.float32),
            pltpu.VMEM((CHUNK, HH), jnp.float32),
            pltpu.VMEM_SHARED((NPAD, HH), jnp.float32),
        ] + [pltpu.SemaphoreType.DMA] * (_SRING + 2),
    )
    def msg_kernel(g_hbm, src_hbm, dst_hbm, z_hbm, out_hbm,
                   didx, si0, si1, si2, si3, rows0, rows1, acc_sh,
                   ss0, ss1, ss2, ss3, sem0, sem1):
        c = lax.axis_index("c")
        s = lax.axis_index("s")
        w = c * NS + s
        base0 = w * (CPT * CHUNK)
        pltpu.sync_copy(z_hbm.at[pl.ds(s * RPT, RPT)], acc_sh.at[pl.ds(s * RPT, RPT)])
        pltpu.sync_copy(dst_hbm.at[pl.ds(w * CPT, CPT)], didx)
        plsc.subcore_barrier()
        sidx = [si0, si1, si2, si3]
        ssems = [ss0, ss1, ss2, ss3]
        rows = [rows0, rows1]
        sems = [sem0, sem1]

        def si_src(j):
            return src_hbm.at[pl.ds(base0 + j * CHUNK, CHUNK)]

        for k in range(_SRING - 1):
            pltpu.async_copy(si_src(k), sidx[k], ssems[k])
        pltpu.make_async_copy(si_src(0), sidx[0], ssems[0]).wait()
        pltpu.async_copy(g_hbm.at[sidx[0]], rows[0], sems[0])

        def body(g_, carry):
            j0 = g_ * _SRING
            for b in range(_SRING):
                j = j0 + b
                rb = b % 2
                nrb = 1 - rb
                nsb = (b + _SRING - 1) % _SRING

                @pl.when(j + _SRING - 1 < CPT)
                def _fire_sidx():
                    pltpu.async_copy(si_src(j + _SRING - 1), sidx[nsb],
                                     ssems[nsb])

                @pl.when(j + 1 < CPT)
                def _fire_gather():
                    nb = (b + 1) % _SRING
                    pltpu.make_async_copy(si_src(j + 1), sidx[nb],
                                          ssems[nb]).wait()
                    pltpu.async_copy(g_hbm.at[sidx[nb]], rows[nrb], sems[nrb])

                pltpu.make_async_copy(g_hbm.at[sidx[b]], rows[rb],
                                      sems[rb]).wait()
                pltpu.sync_copy(rows[rb], acc_sh.at[didx.at[j]], add=True)
            return carry

        lax.fori_loop(0, CPT // _SRING, body, 0)
        plsc.subcore_barrier()
        pltpu.sync_copy(acc_sh.at[pl.ds(s * RPT, RPT)],
                        out_hbm.at[pl.ds(c * NPAD + s * RPT, RPT)])

    return msg_kernel(g, src_flat, dst2d, zeros_h)


def _sc_gather(table, idx_all):
    @functools.partial(
        pl.kernel,
        mesh=_sc_mesh(),
        out_type=jax.ShapeDtypeStruct((GIDX, HH), jnp.float32),
        scratch_types=[
            pltpu.VMEM((CHUNK,), jnp.int32),
            pltpu.VMEM((CHUNK,), jnp.int32),
            pltpu.VMEM((CHUNK, HH), jnp.float32),
            pltpu.VMEM((CHUNK, HH), jnp.float32),
            pltpu.SemaphoreType.DMA,
            pltpu.SemaphoreType.DMA,
        ],
    )
    def gather_kernel(tab_hbm, idx_hbm, out_hbm, idx0, idx1, rows0, rows1,
                      sem0, sem1):
        c = lax.axis_index("c")
        s = lax.axis_index("s")
        base0 = (c * NS + s) * GPT
        idx = [idx0, idx1]
        rows = [rows0, rows1]
        sems = [sem0, sem1]
        pltpu.sync_copy(idx_hbm.at[pl.ds(base0, CHUNK)], idx[0])
        pltpu.async_copy(tab_hbm.at[idx[0]], rows[0], sems[0])

        def body(g_, carry):
            j0 = g_ * 2
            for b in range(2):
                j = j0 + b
                nb = 1 - b

                @pl.when(j + 1 < GCH)
                def _fire():
                    bn = base0 + (j + 1) * CHUNK
                    pltpu.sync_copy(idx_hbm.at[pl.ds(bn, CHUNK)], idx[nb])
                    pltpu.async_copy(tab_hbm.at[idx[nb]], rows[nb], sems[nb])

                pltpu.make_async_copy(tab_hbm.at[idx[b]], rows[b], sems[b]).wait()
                pltpu.sync_copy(rows[b], out_hbm.at[pl.ds(base0 + j * CHUNK, CHUNK)])
            return carry

        lax.fori_loop(0, GCH // 2, body, 0)

    return gather_kernel(table, idx_all)


_ROWS = 1024


def _dot(a, b):
    return jnp.dot(a, b, preferred_element_type=jnp.float32,
                   precision=lax.Precision.HIGHEST)


def _tc_g_dinv(deg_pair, x_pad, W):
    def body(deg_ref, x_ref, w_ref, g_ref, dv_ref):
        dinv = lax.rsqrt(deg_ref[0] + deg_ref[1] + 1.0)
        g_ref[...] = _dot(x_ref[...], w_ref[...]) * dinv
        dv_ref[...] = dinv

    return pl.pallas_call(
        body,
        grid=(NPAD // _ROWS,),
        in_specs=[
            pl.BlockSpec((2, _ROWS, DW), lambda r: (0, r, 0)),
            pl.BlockSpec((_ROWS, HH), lambda r: (r, 0)),
            pl.BlockSpec((HH, HH), lambda r: (0, 0)),
        ],
        out_specs=[pl.BlockSpec((_ROWS, HH), lambda r: (r, 0))] * 2,
        out_shape=[jax.ShapeDtypeStruct((NPAD, HH), jnp.float32)] * 2,
    )(deg_pair, x_pad, W)


def _tc_layer2(p, g, dinvb, b2, W):
    def body(p_ref, g_ref, dv_ref, b_ref, w_ref, o_ref):
        acc = p_ref[0] + p_ref[1] + g_ref[...]
        fv = jnp.maximum(acc * dv_ref[...] + b_ref[...], 0.0)
        o_ref[...] = _dot(fv, w_ref[...]) * dv_ref[...]

    return pl.pallas_call(
        body,
        grid=(NPAD // _ROWS,),
        in_specs=[
            pl.BlockSpec((2, _ROWS, HH), lambda r: (0, r, 0)),
            pl.BlockSpec((_ROWS, HH), lambda r: (r, 0)),
            pl.BlockSpec((_ROWS, HH), lambda r: (r, 0)),
            pl.BlockSpec((1, HH), lambda r: (0, 0)),
            pl.BlockSpec((HH, HH), lambda r: (0, 0)),
        ],
        out_specs=pl.BlockSpec((_ROWS, HH), lambda r: (r, 0)),
        out_shape=jax.ShapeDtypeStruct((NPAD, HH), jnp.float32),
    )(p, g, dinvb, b2, W)


def _tc_out(p, g2, dinvb, b2):
    def body(p_ref, g_ref, dv_ref, b_ref, o_ref):
        o_ref[...] = ((p_ref[0] + p_ref[1] + g_ref[...]) * dv_ref[...]
                      + b_ref[...])

    return pl.pallas_call(
        body,
        grid=(NPAD // _ROWS,),
        in_specs=[
            pl.BlockSpec((2, _ROWS, HH), lambda r: (0, r, 0)),
            pl.BlockSpec((_ROWS, HH), lambda r: (r, 0)),
            pl.BlockSpec((_ROWS, HH), lambda r: (r, 0)),
            pl.BlockSpec((1, HH), lambda r: (0, 0)),
        ],
        out_specs=pl.BlockSpec((_ROWS, HH), lambda r: (r, 0)),
        out_shape=jax.ShapeDtypeStruct((NPAD, HH), jnp.float32),
    )(p, g2, dinvb, b2)


def _tc_resize(ctx_flat, lab, W_resize, br2):
    R = 1024

    def body(ctx_ref, lab_ref, w_ref, b_ref, o_ref):
        o_ref[...] = lab_ref[...] * (_dot(ctx_ref[...], w_ref[...]) + b_ref[...])

    return pl.pallas_call(
        body,
        grid=(LL // R,),
        in_specs=[
            pl.BlockSpec((R, CTX * HH), lambda r: (r, 0)),
            pl.BlockSpec((R, HH), lambda r: (r, 0)),
            pl.BlockSpec((CTX * HH, HH), lambda r: (0, 0)),
            pl.BlockSpec((1, HH), lambda r: (0, 0)),
        ],
        out_specs=pl.BlockSpec((R, HH), lambda r: (r, 0)),
        out_shape=jax.ShapeDtypeStruct((LL, HH), jnp.float32),
    )(ctx_flat, lab, W_resize, br2)


def kernel(x, edge_index, labeled_idx, context_idx, W_gcn, b_gcn, W_resize, b_resize):
    src = edge_index[0]
    dst = edge_index[1]
    pad_n = EPAD - EE
    fill = NN + (jnp.arange(pad_n, dtype=jnp.int32) % (NPAD - NN))
    src_f = jnp.concatenate([src, fill])
    dst_f = jnp.concatenate([dst, fill])
    x_pad = jnp.pad(x, ((0, NPAD - NN), (0, 0)))
    zH = jnp.zeros((NPAD, HH), jnp.float32)
    zND = jnp.zeros((NPAD, DW), jnp.float32)
    onesD = jnp.ones((CHUNK, DW), jnp.float32)
    b2 = b_gcn.reshape(1, HH)
    br2 = b_resize.reshape(1, HH)

    dst2 = dst_f.reshape(NW * CPT, CHUNK)

    deg_pair = _sc_degree(dst_f, onesD, zND).reshape(NC, NPAD, DW)
    g, dinvb = _tc_g_dinv(deg_pair, x_pad, W_gcn)
    p1 = _sc_scatter(g, src_f, dst2, zH).reshape(NC, NPAD, HH)
    g2 = _tc_layer2(p1, g, dinvb, b2, W_gcn)
    p2 = _sc_scatter(g2, src_f, dst2, zH).reshape(NC, NPAD, HH)
    out2 = _tc_out(p2, g2, dinvb, b2)
    idx_all = jnp.concatenate([context_idx.reshape(-1), labeled_idx])
    rows = _sc_gather(out2, idx_all)
    ctx_flat = rows[:LL * CTX].reshape(LL, CTX * HH)
    lab = rows[LL * CTX:]
    return _tc_resize(ctx_flat, lab, W_resize, br2)


# speedup vs baseline: 24.1911x; 1.0000x over previous
import functools

import jax
import jax.numpy as jnp
from jax import lax
from jax.experimental import pallas as pl
from jax.experimental.pallas import tpu as pltpu
from jax.experimental.pallas import tpu_sc as plsc

NN = 10000
EE = 320000
HH = 128
CTX = 5
LL = 4096
NC = 2
NS = 16
NW = NC * NS
CHUNK = 128
CPT = 80
EPAD = NW * CPT * CHUNK
NPAD = 10240
RPT = NPAD // NS
GIDX = LL * CTX + LL
GPT = GIDX // NW
GCH = GPT // CHUNK


def _sc_mesh():
    return plsc.VectorSubcoreMesh(
        core_axis_name="c", subcore_axis_name="s", num_cores=NC, num_subcores=NS)


DW = HH


def _sc_degree(dst_flat, ones_row, zeros_nd):
    @functools.partial(
        pl.kernel,
        mesh=_sc_mesh(),
        out_type=jax.ShapeDtypeStruct((NC * NPAD, DW), jnp.float32),
        scratch_types=[
            pltpu.VMEM((CHUNK,), jnp.int32),
            pltpu.VMEM((CHUNK,), jnp.int32),
            pltpu.VMEM((CHUNK, DW), jnp.float32),
            pltpu.VMEM_SHARED((NPAD, DW), jnp.float32),
            pltpu.SemaphoreType.DMA,
            pltpu.SemaphoreType.DMA,
        ],
    )
    def deg_kernel(dst_hbm, ones_hbm, z_hbm, out_hbm, idx0, idx1, ones_v,
                   acc_sh, sem0, sem1):
        c = lax.axis_index("c")
        s = lax.axis_index("s")
        pltpu.sync_copy(z_hbm.at[pl.ds(s * RPT, RPT)],
                        acc_sh.at[pl.ds(s * RPT, RPT)])
        pltpu.sync_copy(ones_hbm, ones_v)
        plsc.subcore_barrier()
        base0 = (c * NS + s) * (CPT * CHUNK)
        idx = [idx0, idx1]
        sems = [sem0, sem1]
        pltpu.async_copy(dst_hbm.at[pl.ds(base0, CHUNK)], idx[0], sems[0])

        def body(g, carry):
            j0 = g * 2
            for b in range(2):
                j = j0 + b
                nb = 1 - b

                @pl.when(j + 1 < CPT)
                def _fire():
                    pltpu.async_copy(
                        dst_hbm.at[pl.ds(base0 + (j + 1) * CHUNK, CHUNK)],
                        idx[nb], sems[nb])

                pltpu.make_async_copy(
                    dst_hbm.at[pl.ds(base0 + j * CHUNK, CHUNK)],
                    idx[b], sems[b]).wait()
                pltpu.sync_copy(ones_v, acc_sh.at[idx[b]], add=True)
            return carry

        lax.fori_loop(0, CPT // 2, body, 0)
        plsc.subcore_barrier()
        pltpu.sync_copy(acc_sh.at[pl.ds(s * RPT, RPT)],
                        out_hbm.at[pl.ds(c * NPAD + s * RPT, RPT)])

    return deg_kernel(dst_flat, ones_row, zeros_nd)


_SRING = 4


def _sc_scatter(g, src_flat, dst2d, zeros_h):
    @functools.partial(
        pl.kernel,
        mesh=_sc_mesh(),
        out_type=jax.ShapeDtypeStruct((NC * NPAD, HH), jnp.float32),
        scratch_types=[
            pltpu.VMEM((CPT, CHUNK), jnp.int32),
        ] + [pltpu.VMEM((CHUNK,), jnp.int32)] * _SRING + [
            pltpu.VMEM((CHUNK, HH), jnp.float32),
            pltpu.VMEM((CHUNK, HH), jnp.float32),
            pltpu.VMEM_SHARED((NPAD, HH), jnp.float32),
        ] + [pltpu.SemaphoreType.DMA] * (_SRING + 4),
    )
    def msg_kernel(g_hbm, src_hbm, dst_hbm, z_hbm, out_hbm,
                   didx, si0, si1, si2, si3, rows0, rows1, acc_sh,
                   ss0, ss1, ss2, ss3, sem0, sem1, ts0, ts1):
        c = lax.axis_index("c")
        s = lax.axis_index("s")
        w = c * NS + s
        base0 = w * (CPT * CHUNK)
        pltpu.sync_copy(z_hbm.at[pl.ds(s * RPT, RPT)], acc_sh.at[pl.ds(s * RPT, RPT)])
        pltpu.sync_copy(dst_hbm.at[pl.ds(w * CPT, CPT)], didx)
        plsc.subcore_barrier()
        sidx = [si0, si1, si2, si3]
        ssems = [ss0, ss1, ss2, ss3]
        rows = [rows0, rows1]
        sems = [sem0, sem1]
        tsems = [ts0, ts1]

        def si_src(j):
            return src_hbm.at[pl.ds(base0 + j * CHUNK, CHUNK)]

        for k in range(_SRING - 1):
            pltpu.async_copy(si_src(k), sidx[k], ssems[k])
        pltpu.make_async_copy(si_src(0), sidx[0], ssems[0]).wait()
        pltpu.async_copy(g_hbm.at[sidx[0]], rows[0], sems[0])

        def body(g_, carry):
            j0 = g_ * _SRING
            for b in range(_SRING):
                j = j0 + b
                rb = b % 2
                nrb = 1 - rb
                nsb = (b + _SRING - 1) % _SRING

                @pl.when(j + _SRING - 1 < CPT)
                def _fire_sidx():
                    pltpu.async_copy(si_src(j + _SRING - 1), sidx[nsb],
                                     ssems[nsb])

                @pl.when(j + 1 < CPT)
                def _fire_gather():
                    @pl.when(j >= 1)
                    def _drain_prev_scatter():
                        pltpu.make_async_copy(
                            rows[nrb], acc_sh.at[didx.at[j - 1]],
                            tsems[nrb]).wait()

                    nb = (b + 1) % _SRING
                    pltpu.make_async_copy(si_src(j + 1), sidx[nb],
                                          ssems[nb]).wait()
                    pltpu.async_copy(g_hbm.at[sidx[nb]], rows[nrb], sems[nrb])

                pltpu.make_async_copy(g_hbm.at[sidx[b]], rows[rb],
                                      sems[rb]).wait()
                pltpu.async_copy(rows[rb], acc_sh.at[didx.at[j]], tsems[rb],
                                 add=True)
            return carry

        lax.fori_loop(0, CPT // _SRING, body, 0)
        pltpu.make_async_copy(rows[0], acc_sh.at[didx.at[CPT - 2]],
                              tsems[0]).wait()
        pltpu.make_async_copy(rows[1], acc_sh.at[didx.at[CPT - 1]],
                              tsems[1]).wait()
        plsc.subcore_barrier()
        pltpu.sync_copy(acc_sh.at[pl.ds(s * RPT, RPT)],
                        out_hbm.at[pl.ds(c * NPAD + s * RPT, RPT)])

    return msg_kernel(g, src_flat, dst2d, zeros_h)


def _sc_gather(table, idx_all):
    @functools.partial(
        pl.kernel,
        mesh=_sc_mesh(),
        out_type=jax.ShapeDtypeStruct((GIDX, HH), jnp.float32),
        scratch_types=[
            pltpu.VMEM((CHUNK,), jnp.int32),
            pltpu.VMEM((CHUNK,), jnp.int32),
            pltpu.VMEM((CHUNK, HH), jnp.float32),
            pltpu.VMEM((CHUNK, HH), jnp.float32),
            pltpu.SemaphoreType.DMA,
            pltpu.SemaphoreType.DMA,
        ],
    )
    def gather_kernel(tab_hbm, idx_hbm, out_hbm, idx0, idx1, rows0, rows1,
                      sem0, sem1):
        c = lax.axis_index("c")
        s = lax.axis_index("s")
        base0 = (c * NS + s) * GPT
        idx = [idx0, idx1]
        rows = [rows0, rows1]
        sems = [sem0, sem1]
        pltpu.sync_copy(idx_hbm.at[pl.ds(base0, CHUNK)], idx[0])
        pltpu.async_copy(tab_hbm.at[idx[0]], rows[0], sems[0])

        def body(g_, carry):
            j0 = g_ * 2
            for b in range(2):
                j = j0 + b
                nb = 1 - b

                @pl.when(j + 1 < GCH)
                def _fire():
                    bn = base0 + (j + 1) * CHUNK
                    pltpu.sync_copy(idx_hbm.at[pl.ds(bn, CHUNK)], idx[nb])
                    pltpu.async_copy(tab_hbm.at[idx[nb]], rows[nb], sems[nb])

                pltpu.make_async_copy(tab_hbm.at[idx[b]], rows[b], sems[b]).wait()
                pltpu.sync_copy(rows[b], out_hbm.at[pl.ds(base0 + j * CHUNK, CHUNK)])
            return carry

        lax.fori_loop(0, GCH // 2, body, 0)

    return gather_kernel(table, idx_all)


_ROWS = 1024


def _dot(a, b):
    return jnp.dot(a, b, preferred_element_type=jnp.float32,
                   precision=lax.Precision.HIGHEST)


def _tc_g_dinv(deg_pair, x_pad, W):
    def body(deg_ref, x_ref, w_ref, g_ref, dv_ref):
        dinv = lax.rsqrt(deg_ref[0] + deg_ref[1] + 1.0)
        g_ref[...] = _dot(x_ref[...], w_ref[...]) * dinv
        dv_ref[...] = dinv

    return pl.pallas_call(
        body,
        grid=(NPAD // _ROWS,),
        in_specs=[
            pl.BlockSpec((2, _ROWS, DW), lambda r: (0, r, 0)),
            pl.BlockSpec((_ROWS, HH), lambda r: (r, 0)),
            pl.BlockSpec((HH, HH), lambda r: (0, 0)),
        ],
        out_specs=[pl.BlockSpec((_ROWS, HH), lambda r: (r, 0))] * 2,
        out_shape=[jax.ShapeDtypeStruct((NPAD, HH), jnp.float32)] * 2,
    )(deg_pair, x_pad, W)


def _tc_layer2(p, g, dinvb, b2, W):
    def body(p_ref, g_ref, dv_ref, b_ref, w_ref, o_ref):
        acc = p_ref[0] + p_ref[1] + g_ref[...]
        fv = jnp.maximum(acc * dv_ref[...] + b_ref[...], 0.0)
        o_ref[...] = _dot(fv, w_ref[...]) * dv_ref[...]

    return pl.pallas_call(
        body,
        grid=(NPAD // _ROWS,),
        in_specs=[
            pl.BlockSpec((2, _ROWS, HH), lambda r: (0, r, 0)),
            pl.BlockSpec((_ROWS, HH), lambda r: (r, 0)),
            pl.BlockSpec((_ROWS, HH), lambda r: (r, 0)),
            pl.BlockSpec((1, HH), lambda r: (0, 0)),
            pl.BlockSpec((HH, HH), lambda r: (0, 0)),
        ],
        out_specs=pl.BlockSpec((_ROWS, HH), lambda r: (r, 0)),
        out_shape=jax.ShapeDtypeStruct((NPAD, HH), jnp.float32),
    )(p, g, dinvb, b2, W)


def _tc_out(p, g2, dinvb, b2):
    def body(p_ref, g_ref, dv_ref, b_ref, o_ref):
        o_ref[...] = ((p_ref[0] + p_ref[1] + g_ref[...]) * dv_ref[...]
                      + b_ref[...])

    return pl.pallas_call(
        body,
        grid=(NPAD // _ROWS,),
        in_specs=[
            pl.BlockSpec((2, _ROWS, HH), lambda r: (0, r, 0)),
            pl.BlockSpec((_ROWS, HH), lambda r: (r, 0)),
            pl.BlockSpec((_ROWS, HH), lambda r: (r, 0)),
            pl.BlockSpec((1, HH), lambda r: (0, 0)),
        ],
        out_specs=pl.BlockSpec((_ROWS, HH), lambda r: (r, 0)),
        out_shape=jax.ShapeDtypeStruct((NPAD, HH), jnp.float32),
    )(p, g2, dinvb, b2)


def _tc_resize(ctx_flat, lab, W_resize, br2):
    R = 1024

    def body(ctx_ref, lab_ref, w_ref, b_ref, o_ref):
        o_ref[...] = lab_ref[...] * (_dot(ctx_ref[...], w_ref[...]) + b_ref[...])

    return pl.pallas_call(
        body,
        grid=(LL // R,),
        in_specs=[
            pl.BlockSpec((R, CTX * HH), lambda r: (r, 0)),
            pl.BlockSpec((R, HH), lambda r: (r, 0)),
            pl.BlockSpec((CTX * HH, HH), lambda r: (0, 0)),
            pl.BlockSpec((1, HH), lambda r: (0, 0)),
        ],
        out_specs=pl.BlockSpec((R, HH), lambda r: (r, 0)),
        out_shape=jax.ShapeDtypeStruct((LL, HH), jnp.float32),
    )(ctx_flat, lab, W_resize, br2)


def kernel(x, edge_index, labeled_idx, context_idx, W_gcn, b_gcn, W_resize, b_resize):
    src = edge_index[0]
    dst = edge_index[1]
    pad_n = EPAD - EE
    fill = NN + (jnp.arange(pad_n, dtype=jnp.int32) % (NPAD - NN))
    src_f = jnp.concatenate([src, fill])
    dst_f = jnp.concatenate([dst, fill])
    x_pad = jnp.pad(x, ((0, NPAD - NN), (0, 0)))
    zH = jnp.zeros((NPAD, HH), jnp.float32)
    zND = jnp.zeros((NPAD, DW), jnp.float32)
    onesD = jnp.ones((CHUNK, DW), jnp.float32)
    b2 = b_gcn.reshape(1, HH)
    br2 = b_resize.reshape(1, HH)

    dst2 = dst_f.reshape(NW * CPT, CHUNK)

    deg_pair = _sc_degree(dst_f, onesD, zND).reshape(NC, NPAD, DW)
    g, dinvb = _tc_g_dinv(deg_pair, x_pad, W_gcn)
    p1 = _sc_scatter(g, src_f, dst2, zH).reshape(NC, NPAD, HH)
    g2 = _tc_layer2(p1, g, dinvb, b2, W_gcn)
    p2 = _sc_scatter(g2, src_f, dst2, zH).reshape(NC, NPAD, HH)
    out2 = _tc_out(p2, g2, dinvb, b2)
    idx_all = jnp.concatenate([context_idx.reshape(-1), labeled_idx])
    rows = _sc_gather(out2, idx_all)
    ctx_flat = rows[:LL * CTX].reshape(LL, CTX * HH)
    lab = rows[LL * CTX:]
    return _tc_resize(ctx_flat, lab, W_resize, br2)
